# Initial kernel scaffold; baseline (speedup 1.0000x reference)
import jax
import jax.numpy as jnp
from jax.experimental import pallas as pl


def kernel(X, edge_index, W1, b1, W2, b2, W3, b3, W4, b4):
    raise NotImplementedError("write your pallas kernel here")


# speedup vs baseline: 3.7515x; 3.7515x over previous
import jax
import jax.numpy as jnp
from jax import lax
from jax.experimental import pallas as pl
from jax.experimental.pallas import tpu as pltpu
from jax.experimental.pallas import tpu_sc as plsc

N = 10000
E = 320000
D = 128
H1 = 64
H2 = 128
HFC = 256

NC = 2
NS = 16
HALF = 5120
AR = 5248
RPT = AR // NS
P = 2 * HALF
CH = 128
NCHUNK = E // CH
ROUNDS = -(-NCHUNK // NS)


def _make_edge_pass(layer1):
    mesh = plsc.VectorSubcoreMesh(core_axis_name="c", subcore_axis_name="s")
    out_type = jax.ShapeDtypeStruct((NC, AR, 128), jnp.float32)
    scratch = [
        pltpu.VMEM((CH,), jnp.int32),
        pltpu.VMEM((CH,), jnp.int32),
        pltpu.VMEM((CH,), jnp.int32),
        pltpu.VMEM((CH, 128), jnp.float32),
        pltpu.VMEM((CH, 128), jnp.float32),
        pltpu.VMEM((CH, 128), jnp.float32),
        pltpu.VMEM_SHARED((AR, 128), jnp.float32),
        pltpu.SemaphoreType.DMA,
        pltpu.SemaphoreType.DMA,
    ]

    def body(a_hbm, b_hbm, ei_hbm, zz_hbm, out_hbm,
             dstv, srcv, dstl, rd, rs, mb, acc, sem_a, sem_b):
        c = lax.axis_index("c")
        s = lax.axis_index("s")
        base_row = s * RPT
        pltpu.sync_copy(zz_hbm.at[pl.ds(base_row, RPT)],
                        acc.at[pl.ds(base_row, RPT)])
        if layer1:
            def onerow(r, _):
                for q in range(4, 8):
                    mb[r, pl.ds(q * 16, 16)] = jnp.ones((16,), jnp.float32)
                return 0
            lax.fori_loop(0, CH, onerow, 0)
        plsc.subcore_barrier()
        rbase = c * HALF

        def chunk(t, _):
            k = t * NS + s

            @pl.when(k < NCHUNK)
            def _():
                off = pl.multiple_of(k * CH, 8)
                pltpu.sync_copy(ei_hbm.at[1, pl.ds(off, CH)], dstv)
                pltpu.sync_copy(ei_hbm.at[0, pl.ds(off, CH)], srcv)
                cpa = pltpu.async_copy(a_hbm.at[dstv], rd, sem_a)
                cpb = pltpu.async_copy(b_hbm.at[srcv], rs, sem_b)
                cpa.wait()
                cpb.wait()

                def row(r, _):
                    if layer1:
                        for q in range(4):
                            va = rd[r, pl.ds(q * 16, 16)]
                            vb = rs[r, pl.ds(64 + q * 16, 16)]
                            mb[r, pl.ds(q * 16, 16)] = jnp.maximum(va + vb, 0.0)
                    else:
                        for q in range(8):
                            va = rd[r, pl.ds(q * 16, 16)]
                            vb = rs[r, pl.ds(q * 16, 16)]
                            mb[r, pl.ds(q * 16, 16)] = jnp.maximum(va + vb, 0.0)
                    return 0
                lax.fori_loop(0, CH, row, 0)
                for q in range(CH // 16):
                    l = dstv[pl.ds(q * 16, 16)] - rbase
                    ok = (l >= 0) & (l < HALF)
                    dstl[pl.ds(q * 16, 16)] = jnp.where(ok, l, HALF)
                pltpu.sync_copy(mb, acc.at[dstl], add=True)
            return 0
        lax.fori_loop(0, ROUNDS, chunk, 0)
        plsc.subcore_barrier()
        pltpu.sync_copy(acc.at[pl.ds(base_row, RPT)],
                        out_hbm.at[c, pl.ds(base_row, RPT)])

    return pl.kernel(body, out_type=out_type, mesh=mesh,
                     scratch_types=scratch)


_edge1 = _make_edge_pass(layer1=True)
_edge2 = _make_edge_pass(layer1=False)


def _proj1_body(x_ref, w_ref, b_ref, ab_ref):
    y = jnp.dot(x_ref[...], w_ref[...], preferred_element_type=jnp.float32)
    ab_ref[...] = y + b_ref[...]


_proj1 = pl.pallas_call(
    _proj1_body,
    out_shape=jax.ShapeDtypeStruct((N, 2 * H1), jnp.float32),
)


def _sums_counts(s_ref):
    hcat = jnp.concatenate([s_ref[0, :HALF], s_ref[1, :HALF]], axis=0)
    cnt = jnp.maximum(hcat[:, H1], 1.0)
    return hcat[:, :H1] / cnt[:, None], cnt


def _mid_body(s_ref, w_ref, b_ref, a_ref, bo_ref):
    h, _ = _sums_counts(s_ref)
    y = jnp.dot(h, w_ref[...], preferred_element_type=jnp.float32)
    a_ref[...] = y[:, :H2] + b_ref[...]
    bo_ref[...] = y[:, H2:]


_mid = pl.pallas_call(
    _mid_body,
    out_shape=(jax.ShapeDtypeStruct((P, H2), jnp.float32),
               jax.ShapeDtypeStruct((P, H2), jnp.float32)),
)


def _final_body(s1_ref, s2_ref, w3_ref, b3_ref, w4_ref, b4_ref, o_ref):
    _, cnt = _sums_counts(s1_ref)
    h2 = jnp.concatenate([s2_ref[0, :HALF], s2_ref[1, :HALF]], axis=0)
    h2 = h2 / cnt[:, None]
    f = jnp.dot(h2, w3_ref[...], preferred_element_type=jnp.float32)
    f = jnp.maximum(f + b3_ref[...], 0.0)
    z = jnp.dot(f, w4_ref[...], preferred_element_type=jnp.float32)
    z = z + b4_ref[...]
    o_ref[...] = 1.0 / (1.0 + jnp.exp(-z))


_final = pl.pallas_call(
    _final_body,
    out_shape=jax.ShapeDtypeStruct((P, 1), jnp.float32),
)


def kernel(X, edge_index, W1, b1, W2, b2, W3, b3, W4, b4):
    W1a, W1b = W1[:, :D], W1[:, D:]
    Wc1 = jnp.concatenate([(W1a - W1b).T, W1b.T], axis=1)
    b1p = jnp.concatenate([b1, jnp.zeros((H1,), jnp.float32)])
    W2a, W2b = W2[:, :H1], W2[:, H1:]
    Wc2 = jnp.concatenate([(W2a - W2b).T, W2b.T], axis=1)
    zz = jnp.zeros((AR, 128), jnp.float32)

    AB1 = _proj1(X, Wc1, b1p)
    S1 = _edge1(AB1, AB1, edge_index, zz)
    A2, B2 = _mid(S1, Wc2, b2)
    S2 = _edge2(A2, B2, edge_index, zz)
    out = _final(S1, S2, W3.T, b3, W4.T, b4)
    return out[:N, 0]


# speedup vs baseline: 5.9845x; 1.5952x over previous
import jax
import jax.numpy as jnp
from jax import lax
from jax.experimental import pallas as pl
from jax.experimental.pallas import tpu as pltpu
from jax.experimental.pallas import tpu_sc as plsc

N = 10000
E = 320000
D = 128
H1 = 64
H2 = 128
HFC = 256

NC = 2
NS = 16
HALF = 5120
AR = 5248
RPT = AR // NS
P = 2 * HALF
CH = 128
NCHUNK = E // CH
ROUNDS = -(-NCHUNK // NS)


def _make_edge_pass(layer1):
    mesh = plsc.VectorSubcoreMesh(core_axis_name="c", subcore_axis_name="s")
    out_type = jax.ShapeDtypeStruct((NC, AR, 128), jnp.float32)
    scratch = [
        pltpu.VMEM((CH,), jnp.int32),
        pltpu.VMEM((CH,), jnp.int32),
        pltpu.VMEM((CH,), jnp.int32),
        pltpu.VMEM((CH,), jnp.int32),
        pltpu.VMEM((CH,), jnp.int32),
        pltpu.VMEM((CH, 128), jnp.float32),
        pltpu.VMEM((CH, 128), jnp.float32),
        pltpu.VMEM((CH, 128), jnp.float32),
        pltpu.VMEM((CH, 128), jnp.float32),
        pltpu.VMEM((CH, 128), jnp.float32),
        pltpu.VMEM_SHARED((AR, 128), jnp.float32),
        pltpu.SemaphoreType.DMA,
        pltpu.SemaphoreType.DMA,
        pltpu.SemaphoreType.DMA,
        pltpu.SemaphoreType.DMA,
    ]

    def body(a_hbm, b_hbm, ei_hbm, zz_hbm, out_hbm,
             dst0, dst1, src0, src1, dstl, rd0, rd1, rs0, rs1, mb, acc,
             sa0, sa1, sb0, sb1):
        dstv = (dst0, dst1)
        srcv = (src0, src1)
        rd = (rd0, rd1)
        rs = (rs0, rs1)
        sem_a = (sa0, sa1)
        sem_b = (sb0, sb1)
        c = lax.axis_index("c")
        s = lax.axis_index("s")
        base_row = s * RPT
        pltpu.sync_copy(zz_hbm.at[pl.ds(base_row, RPT)],
                        acc.at[pl.ds(base_row, RPT)])
        if layer1:
            def onerow(r, _):
                for q in range(4, 8):
                    mb[r, pl.ds(q * 16, 16)] = jnp.ones((16,), jnp.float32)
                return 0
            lax.fori_loop(0, CH, onerow, 0)
        plsc.subcore_barrier()
        rbase = c * HALF

        def issue(b, k):
            @pl.when(k < NCHUNK)
            def _():
                off = pl.multiple_of(k * CH, 8)
                pltpu.sync_copy(ei_hbm.at[1, pl.ds(off, CH)], dstv[b])
                pltpu.sync_copy(ei_hbm.at[0, pl.ds(off, CH)], srcv[b])
                pltpu.async_copy(a_hbm.at[dstv[b]], rd[b], sem_a[b])
                pltpu.async_copy(b_hbm.at[srcv[b]], rs[b], sem_b[b])

        def process(b, k):
            @pl.when(k < NCHUNK)
            def _():
                pltpu.make_async_copy(a_hbm.at[dstv[b]], rd[b],
                                      sem_a[b]).wait()
                pltpu.make_async_copy(b_hbm.at[srcv[b]], rs[b],
                                      sem_b[b]).wait()

                def row(r, _):
                    if layer1:
                        for q in range(4):
                            va = rd[b][r, pl.ds(q * 16, 16)]
                            vb = rs[b][r, pl.ds(64 + q * 16, 16)]
                            mb[r, pl.ds(q * 16, 16)] = jnp.maximum(va + vb, 0.0)
                    else:
                        for q in range(8):
                            va = rd[b][r, pl.ds(q * 16, 16)]
                            vb = rs[b][r, pl.ds(q * 16, 16)]
                            mb[r, pl.ds(q * 16, 16)] = jnp.maximum(va + vb, 0.0)
                    return 0
                lax.fori_loop(0, CH, row, 0)
                for q in range(CH // 16):
                    l = dstv[b][pl.ds(q * 16, 16)] - rbase
                    ok = (l >= 0) & (l < HALF)
                    dstl[pl.ds(q * 16, 16)] = jnp.where(ok, l, HALF)
                pltpu.sync_copy(mb, acc.at[dstl], add=True)

        issue(0, s)

        def pair(g, _):
            for b in (0, 1):
                u = 2 * g + b
                k = u * NS + s
                issue(1 - b, k + NS)
                process(b, k)
            return 0
        lax.fori_loop(0, (ROUNDS + 1) // 2, pair, 0)
        plsc.subcore_barrier()
        pltpu.sync_copy(acc.at[pl.ds(base_row, RPT)],
                        out_hbm.at[c, pl.ds(base_row, RPT)])

    return pl.kernel(body, out_type=out_type, mesh=mesh,
                     scratch_types=scratch)


_edge1 = _make_edge_pass(layer1=True)
_edge2 = _make_edge_pass(layer1=False)


def _proj1_body(x_ref, w_ref, b_ref, ab_ref):
    y = jnp.dot(x_ref[...], w_ref[...], preferred_element_type=jnp.float32)
    ab_ref[...] = y + b_ref[...]


_proj1 = pl.pallas_call(
    _proj1_body,
    out_shape=jax.ShapeDtypeStruct((N, 2 * H1), jnp.float32),
)


def _sums_counts(s_ref):
    hcat = jnp.concatenate([s_ref[0, :HALF], s_ref[1, :HALF]], axis=0)
    cnt = jnp.maximum(hcat[:, H1], 1.0)
    return hcat[:, :H1] / cnt[:, None], cnt


def _mid_body(s_ref, w_ref, b_ref, a_ref, bo_ref):
    h, _ = _sums_counts(s_ref)
    y = jnp.dot(h, w_ref[...], preferred_element_type=jnp.float32)
    a_ref[...] = y[:, :H2] + b_ref[...]
    bo_ref[...] = y[:, H2:]


_mid = pl.pallas_call(
    _mid_body,
    out_shape=(jax.ShapeDtypeStruct((P, H2), jnp.float32),
               jax.ShapeDtypeStruct((P, H2), jnp.float32)),
)


def _final_body(s1_ref, s2_ref, w3_ref, b3_ref, w4_ref, b4_ref, o_ref):
    _, cnt = _sums_counts(s1_ref)
    h2 = jnp.concatenate([s2_ref[0, :HALF], s2_ref[1, :HALF]], axis=0)
    h2 = h2 / cnt[:, None]
    f = jnp.dot(h2, w3_ref[...], preferred_element_type=jnp.float32)
    f = jnp.maximum(f + b3_ref[...], 0.0)
    z = jnp.dot(f, w4_ref[...], preferred_element_type=jnp.float32)
    z = z + b4_ref[...]
    o_ref[...] = 1.0 / (1.0 + jnp.exp(-z))


_final = pl.pallas_call(
    _final_body,
    out_shape=jax.ShapeDtypeStruct((P, 1), jnp.float32),
)


def kernel(X, edge_index, W1, b1, W2, b2, W3, b3, W4, b4):
    W1a, W1b = W1[:, :D], W1[:, D:]
    Wc1 = jnp.concatenate([(W1a - W1b).T, W1b.T], axis=1)
    b1p = jnp.concatenate([b1, jnp.zeros((H1,), jnp.float32)])
    W2a, W2b = W2[:, :H1], W2[:, H1:]
    Wc2 = jnp.concatenate([(W2a - W2b).T, W2b.T], axis=1)
    zz = jnp.zeros((AR, 128), jnp.float32)

    AB1 = _proj1(X, Wc1, b1p)
    S1 = _edge1(AB1, AB1, edge_index, zz)
    A2, B2 = _mid(S1, Wc2, b2)
    S2 = _edge2(A2, B2, edge_index, zz)
    out = _final(S1, S2, W3.T, b3, W4.T, b4)
    return out[:N, 0]
